# baseline (speedup 1.0000x reference)
import functools

import jax
import jax.numpy as jnp
from jax import lax
from jax.experimental import pallas as pl
from jax.experimental.pallas import tpu as pltpu
from jax.experimental.pallas import tpu_sc as plsc

NC, NS = 2, 16
CHUNKS, CW = 100, 100
STAGE = 25
NBUF = 3
NP = 10240
RB = 1024
EP = NS * CHUNKS * CW


def _sc_mesh():
    return plsc.VectorSubcoreMesh(core_axis_name="c", subcore_axis_name="s")


def _degrees(edges_t, ones_np):
    @functools.partial(
        pl.kernel,
        out_type=jax.ShapeDtypeStruct((2 * NP,), jnp.float32),
        mesh=_sc_mesh(),
        scratch_types=[
            pltpu.VMEM((CHUNKS, CW), jnp.int32),
            pltpu.VMEM((CW,), jnp.float32),
            pltpu.VMEM_SHARED((NP,), jnp.float32),
        ],
    )
    def k(edges_hbm, ones_hbm, deg_hbm, idx_v, ones_v, deg_sh):
        c = lax.axis_index("c")
        s = lax.axis_index("s")
        pltpu.sync_copy(edges_hbm.at[c * NS + s], idx_v)
        pltpu.sync_copy(ones_hbm.at[pl.ds(0, CW)], ones_v)

        @pl.when(s == 0)
        def _():
            pltpu.sync_copy(ones_hbm, deg_sh)

        plsc.subcore_barrier()

        @pl.loop(0, CHUNKS)
        def _(j):
            pltpu.sync_copy(ones_v, deg_sh.at[idx_v.at[j]], add=True)

        plsc.subcore_barrier()

        @pl.when(s == 0)
        def _():
            pltpu.sync_copy(deg_sh, deg_hbm.at[pl.ds(c * NP, NP)])

    return k(edges_t, ones_np)


def _scatter_rows(vals_flat, src2_t, dst_t, d):
    rows_per = NP // NS

    @functools.partial(
        pl.kernel,
        out_type=jax.ShapeDtypeStruct((2 * NP, d), jnp.float32),
        mesh=_sc_mesh(),
        scratch_types=[
            pltpu.VMEM((STAGE, CW), jnp.int32),
            pltpu.VMEM((STAGE, CW), jnp.int32),
            pltpu.VMEM((NBUF, CW, d), jnp.float32),
            pltpu.SemaphoreType.DMA((NBUF,)),
            pltpu.SemaphoreType.DMA((NBUF,)),
            pltpu.VMEM_SHARED((NP, d), jnp.float32),
        ],
    )
    def k(vals_hbm, src_hbm, dst_hbm, out_hbm, src_v, dst_v, rows_v, gsem,
          ssem, acc_sh):
        c = lax.axis_index("c")
        s = lax.axis_index("s")
        base = s * rows_per
        pltpu.sync_copy(vals_hbm.at[pl.ds(c * NP + base, rows_per)],
                        acc_sh.at[pl.ds(base, rows_per)])
        plsc.subcore_barrier()

        def gather(j, b):
            pltpu.async_copy(vals_hbm.at[src_v.at[j]], rows_v.at[b],
                             gsem.at[b])

        def wait_gather(b):
            pltpu.make_async_copy(vals_hbm.at[src_v.at[0]], rows_v.at[b],
                                  gsem.at[b]).wait()

        def scatter(j, b):
            pltpu.async_copy(rows_v.at[b], acc_sh.at[dst_v.at[j]],
                             ssem.at[b], add=True)

        def wait_scatter(b):
            pltpu.make_async_copy(rows_v.at[b], acc_sh.at[dst_v.at[0]],
                                  ssem.at[b]).wait()

        nstage = CHUNKS // STAGE
        for st in range(nstage):
            pltpu.sync_copy(src_hbm.at[(c * NS + s) * nstage + st], src_v)
            pltpu.sync_copy(dst_hbm.at[s * nstage + st], dst_v)
            gather(0, 0)
            gather(1, 1)
            wait_gather(0)
            gather(2, 2)
            scatter(0, 0)

            @pl.loop(1, STAGE, step=NBUF)
            def _(j0):
                for dj in range(NBUF):
                    j = j0 + dj
                    b = (1 + dj) % NBUF
                    b2 = (b + 2) % NBUF
                    wait_gather(b)
                    wait_scatter(b2)

                    @pl.when(j + 2 < STAGE)
                    def _():
                        gather(j + 2, b2)

                    scatter(j, b)

            wait_scatter((STAGE - 1) % NBUF)

        plsc.subcore_barrier()
        pltpu.sync_copy(acc_sh.at[pl.ds(base, rows_per)],
                        out_hbm.at[pl.ds(c * NP + base, rows_per)])

    return k(vals_flat, src2_t, dst_t)


def _prep(x, deg_o):
    n, d = x.shape
    half = d // 2

    def body(x_ref, dg_ref, out_ref):
        do = lax.rsqrt(dg_ref[...])
        xs = x_ref[...] * do
        out_ref[0] = xs[:, :half]
        out_ref[1] = xs[:, half:]

    return pl.pallas_call(
        body,
        grid=(n // RB,),
        in_specs=[pl.BlockSpec((RB, d), lambda i: (i, 0)),
                  pl.BlockSpec((RB, 1), lambda i: (i, 0))],
        out_specs=pl.BlockSpec((2, RB, half), lambda i: (0, i, 0)),
        out_shape=jax.ShapeDtypeStruct((2, n, half), jnp.float32),
    )(x, deg_o)


def _mm(agg2, deg_i, deg_o, W1, b1, W2):
    _, n, half = agg2.shape
    d_hid = W1.shape[1]

    def body(a_ref, di_ref, do_ref, w1_ref, b1_ref, w2_ref, out_ref):
        di = lax.rsqrt(di_ref[...])
        do = lax.rsqrt(do_ref[...])
        agg = jnp.concatenate([a_ref[0], a_ref[1]], axis=1)
        h1 = jnp.dot((agg * di).astype(jnp.bfloat16), w1_ref[...],
                     preferred_element_type=jnp.float32)
        h1 = jnp.maximum(h1 + b1_ref[...], 0.0)
        t = jnp.dot((h1 * do).astype(jnp.bfloat16), w2_ref[...],
                    preferred_element_type=jnp.float32)
        out_ref[0] = t[:, :half]
        out_ref[1] = t[:, half:]

    return pl.pallas_call(
        body,
        grid=(n // RB,),
        in_specs=[pl.BlockSpec((2, RB, half), lambda i: (0, i, 0)),
                  pl.BlockSpec((RB, 1), lambda i: (i, 0)),
                  pl.BlockSpec((RB, 1), lambda i: (i, 0)),
                  pl.BlockSpec(W1.shape, lambda i: (0, 0)),
                  pl.BlockSpec((1, d_hid), lambda i: (0, 0)),
                  pl.BlockSpec(W2.shape, lambda i: (0, 0))],
        out_specs=pl.BlockSpec((2, RB, half), lambda i: (0, i, 0)),
        out_shape=jax.ShapeDtypeStruct((2, n, half), jnp.float32),
    )(agg2, deg_i, deg_o, W1, b1, W2)


def _final(agg2, deg_i, b2, n_valid):
    _, n, half = agg2.shape
    d = 2 * half

    def body(a_ref, di_ref, b2_ref, out_ref):
        i = pl.program_id(0)
        di = lax.rsqrt(di_ref[...])
        agg = jnp.concatenate([a_ref[0], a_ref[1]], axis=1)
        h2 = jnp.maximum(agg * di + b2_ref[...], 0.0)
        row = i * RB + lax.broadcasted_iota(jnp.int32, (RB, 1), 0)
        h2 = jnp.where(row < n_valid, h2, 0.0)
        part = jnp.sum(h2, axis=0, keepdims=True)

        @pl.when(i == 0)
        def _():
            out_ref[...] = jnp.zeros((1, d), jnp.float32)

        out_ref[...] += part

        @pl.when(i == pl.num_programs(0) - 1)
        def _():
            out_ref[...] = out_ref[...] * (1.0 / n_valid)

    return pl.pallas_call(
        body,
        grid=(n // RB,),
        in_specs=[pl.BlockSpec((2, RB, half), lambda i: (0, i, 0)),
                  pl.BlockSpec((RB, 1), lambda i: (i, 0)),
                  pl.BlockSpec((1, d), lambda i: (0, 0))],
        out_specs=pl.BlockSpec((1, d), lambda i: (0, 0)),
        out_shape=jax.ShapeDtypeStruct((1, d), jnp.float32),
    )(agg2, deg_i, b2)


def kernel(x, edge_index, W1, b1, W2, b2):
    n, d_in = x.shape
    half = d_in // 2

    e = edge_index.shape[1]
    pad_e = jnp.full((EP - e,), n, jnp.int32)
    src = jnp.concatenate([edge_index[0], pad_e])
    dst = jnp.concatenate([edge_index[1], pad_e])
    edges_t = jnp.stack([src, dst]).reshape(2 * NS, CHUNKS, CW)
    ones_np = jnp.ones((NP,), jnp.float32)

    deg = _degrees(edges_t, ones_np)
    deg_o = deg[:NP].reshape(NP, 1)
    deg_i = deg[NP:].reshape(NP, 1)

    nstage = CHUNKS // STAGE
    src2 = jnp.stack([src, src + NP]).reshape(2 * NS * nstage, STAGE, CW)
    dst_t = dst.reshape(NS * nstage, STAGE, CW)

    x_pad = jnp.pad(x, ((0, NP - n), (0, 0)))
    xs2 = _prep(x_pad, deg_o)
    agg1 = _scatter_rows(xs2.reshape(2 * NP, half), src2, dst_t, half)
    t2 = _mm(agg1.reshape(2, NP, half), deg_i, deg_o,
             W1.astype(jnp.bfloat16), b1.reshape(1, -1),
             W2.astype(jnp.bfloat16))
    agg2 = _scatter_rows(t2.reshape(2 * NP, half), src2, dst_t, half)
    return _final(agg2.reshape(2, NP, half), deg_i, b2.reshape(1, -1), n)


# speedup vs baseline: 1.0256x; 1.0256x over previous
import functools

import jax
import jax.numpy as jnp
from jax import lax
from jax.experimental import pallas as pl
from jax.experimental.pallas import tpu as pltpu
from jax.experimental.pallas import tpu_sc as plsc

NC, NS = 2, 16
CHUNKS, CW = 100, 100
STAGE = 25
NBUF = 3
NP = 10240
RB = 1000
EP = NS * CHUNKS * CW


def _sc_mesh():
    return plsc.VectorSubcoreMesh(core_axis_name="c", subcore_axis_name="s")


def _degrees(edges_t, ones_np):
    @functools.partial(
        pl.kernel,
        out_type=jax.ShapeDtypeStruct((2 * NP,), jnp.float32),
        mesh=_sc_mesh(),
        scratch_types=[
            pltpu.VMEM((CHUNKS, CW), jnp.int32),
            pltpu.VMEM((CW,), jnp.float32),
            pltpu.VMEM_SHARED((NP,), jnp.float32),
        ],
    )
    def k(edges_hbm, ones_hbm, deg_hbm, idx_v, ones_v, deg_sh):
        c = lax.axis_index("c")
        s = lax.axis_index("s")
        pltpu.sync_copy(edges_hbm.at[c * NS + s], idx_v)
        pltpu.sync_copy(ones_hbm.at[pl.ds(0, CW)], ones_v)

        @pl.when(s == 0)
        def _():
            pltpu.sync_copy(ones_hbm, deg_sh)

        plsc.subcore_barrier()

        @pl.loop(0, CHUNKS)
        def _(j):
            pltpu.sync_copy(ones_v, deg_sh.at[idx_v.at[j]], add=True)

        plsc.subcore_barrier()

        @pl.when(s == 0)
        def _():
            pltpu.sync_copy(deg_sh, deg_hbm.at[pl.ds(c * NP, NP)])

    return k(edges_t, ones_np)


def _scatter_rows(vals_flat, src2_t, dst_t, d):
    rows_per = NP // NS

    @functools.partial(
        pl.kernel,
        out_type=jax.ShapeDtypeStruct((2 * NP, d), jnp.float32),
        mesh=_sc_mesh(),
        scratch_types=[
            pltpu.VMEM((STAGE, CW), jnp.int32),
            pltpu.VMEM((STAGE, CW), jnp.int32),
            pltpu.VMEM((NBUF, CW, d), jnp.float32),
            pltpu.SemaphoreType.DMA((NBUF,)),
            pltpu.SemaphoreType.DMA((NBUF,)),
            pltpu.VMEM_SHARED((NP, d), jnp.float32),
        ],
    )
    def k(vals_hbm, src_hbm, dst_hbm, out_hbm, src_v, dst_v, rows_v, gsem,
          ssem, acc_sh):
        c = lax.axis_index("c")
        s = lax.axis_index("s")
        base = s * rows_per
        pltpu.sync_copy(vals_hbm.at[pl.ds(c * NP + base, rows_per)],
                        acc_sh.at[pl.ds(base, rows_per)])
        plsc.subcore_barrier()

        def gather(j, b):
            pltpu.async_copy(vals_hbm.at[src_v.at[j]], rows_v.at[b],
                             gsem.at[b])

        def wait_gather(b):
            pltpu.make_async_copy(vals_hbm.at[src_v.at[0]], rows_v.at[b],
                                  gsem.at[b]).wait()

        def scatter(j, b):
            pltpu.async_copy(rows_v.at[b], acc_sh.at[dst_v.at[j]],
                             ssem.at[b], add=True)

        def wait_scatter(b):
            pltpu.make_async_copy(rows_v.at[b], acc_sh.at[dst_v.at[0]],
                                  ssem.at[b]).wait()

        nstage = CHUNKS // STAGE
        for st in range(nstage):
            pltpu.sync_copy(src_hbm.at[(c * NS + s) * nstage + st], src_v)
            pltpu.sync_copy(dst_hbm.at[s * nstage + st], dst_v)
            gather(0, 0)
            gather(1, 1)
            wait_gather(0)
            gather(2, 2)
            scatter(0, 0)

            @pl.loop(1, STAGE, step=NBUF)
            def _(j0):
                for dj in range(NBUF):
                    j = j0 + dj
                    b = (1 + dj) % NBUF
                    b2 = (b + 2) % NBUF
                    wait_gather(b)
                    wait_scatter(b2)

                    @pl.when(j + 2 < STAGE)
                    def _():
                        gather(j + 2, b2)

                    scatter(j, b)

            wait_scatter((STAGE - 1) % NBUF)

        plsc.subcore_barrier()
        pltpu.sync_copy(acc_sh.at[pl.ds(base, rows_per)],
                        out_hbm.at[pl.ds(c * NP + base, rows_per)])

    return k(vals_flat, src2_t, dst_t)


def _prep(x, deg_o):
    n, d = x.shape
    half = d // 2

    def body(x_ref, dg_ref, out_ref):
        do = lax.rsqrt(dg_ref[...])
        xs = x_ref[...] * do
        out_ref[0] = xs[:, :half]
        out_ref[1] = xs[:, half:]

    return pl.pallas_call(
        body,
        grid=(n // RB,),
        in_specs=[pl.BlockSpec((RB, d), lambda i: (i, 0)),
                  pl.BlockSpec((RB, 1), lambda i: (i, 0))],
        out_specs=pl.BlockSpec((2, RB, half), lambda i: (0, i, 0)),
        out_shape=jax.ShapeDtypeStruct((2, NP, half), jnp.float32),
    )(x, deg_o)


def _mm(agg2, deg_i, deg_o, W1, b1, W2):
    _, n, half = agg2.shape
    d_hid = W1.shape[1]

    def body(a_ref, di_ref, do_ref, w1_ref, b1_ref, w2_ref, out_ref):
        di = lax.rsqrt(di_ref[...])
        do = lax.rsqrt(do_ref[...])
        agg = jnp.concatenate([a_ref[0], a_ref[1]], axis=1)
        h1 = jnp.dot((agg * di).astype(jnp.bfloat16), w1_ref[...],
                     preferred_element_type=jnp.float32)
        h1 = jnp.maximum(h1 + b1_ref[...], 0.0)
        t = jnp.dot((h1 * do).astype(jnp.bfloat16), w2_ref[...],
                    preferred_element_type=jnp.float32)
        out_ref[0] = t[:, :half]
        out_ref[1] = t[:, half:]

    return pl.pallas_call(
        body,
        grid=(n // RB,),
        in_specs=[pl.BlockSpec((2, RB, half), lambda i: (0, i, 0)),
                  pl.BlockSpec((RB, 1), lambda i: (i, 0)),
                  pl.BlockSpec((RB, 1), lambda i: (i, 0)),
                  pl.BlockSpec(W1.shape, lambda i: (0, 0)),
                  pl.BlockSpec((1, d_hid), lambda i: (0, 0)),
                  pl.BlockSpec(W2.shape, lambda i: (0, 0))],
        out_specs=pl.BlockSpec((2, RB, half), lambda i: (0, i, 0)),
        out_shape=jax.ShapeDtypeStruct((2, n, half), jnp.float32),
    )(agg2, deg_i, deg_o, W1, b1, W2)


def _final(agg2, deg_i, b2, n_valid):
    d = 2 * agg2.shape[2]

    def body(a_ref, di_ref, b2_ref, out_ref):
        i = pl.program_id(0)
        di = lax.rsqrt(di_ref[...])
        agg = jnp.concatenate([a_ref[0], a_ref[1]], axis=1)
        h2 = jnp.maximum(agg * di + b2_ref[...], 0.0)
        part = jnp.sum(h2, axis=0, keepdims=True)

        @pl.when(i == 0)
        def _():
            out_ref[...] = jnp.zeros((1, d), jnp.float32)

        out_ref[...] += part

        @pl.when(i == pl.num_programs(0) - 1)
        def _():
            out_ref[...] = out_ref[...] * (1.0 / n_valid)

    return pl.pallas_call(
        body,
        grid=(n_valid // RB,),
        in_specs=[pl.BlockSpec((2, RB, d // 2), lambda i: (0, i, 0)),
                  pl.BlockSpec((RB, 1), lambda i: (i, 0)),
                  pl.BlockSpec((1, d), lambda i: (0, 0))],
        out_specs=pl.BlockSpec((1, d), lambda i: (0, 0)),
        out_shape=jax.ShapeDtypeStruct((1, d), jnp.float32),
    )(agg2, deg_i, b2)


def kernel(x, edge_index, W1, b1, W2, b2):
    n, d_in = x.shape
    half = d_in // 2

    src = edge_index[0]
    dst = edge_index[1]
    edges_t = edge_index.reshape(2 * NS, CHUNKS, CW)
    ones_np = jnp.ones((NP,), jnp.float32)

    deg = _degrees(edges_t, ones_np)
    deg_o = deg[:NP].reshape(NP, 1)
    deg_i = deg[NP:].reshape(NP, 1)

    nstage = CHUNKS // STAGE
    src2 = jnp.stack([src, src + NP]).reshape(2 * NS * nstage, STAGE, CW)
    dst_t = dst.reshape(NS * nstage, STAGE, CW)

    xs2 = _prep(x, deg_o)
    agg1 = _scatter_rows(xs2.reshape(2 * NP, half), src2, dst_t, half)
    t2 = _mm(agg1.reshape(2, NP, half), deg_i, deg_o,
             W1.astype(jnp.bfloat16), b1.reshape(1, -1),
             W2.astype(jnp.bfloat16))
    agg2 = _scatter_rows(t2.reshape(2 * NP, half), src2, dst_t, half)
    return _final(agg2.reshape(2, NP, half), deg_i, b2.reshape(1, -1), n)


# speedup vs baseline: 1.0296x; 1.0040x over previous
import functools

import jax
import jax.numpy as jnp
from jax import lax
from jax.experimental import pallas as pl
from jax.experimental.pallas import tpu as pltpu
from jax.experimental.pallas import tpu_sc as plsc

NC, NS = 2, 16
CHUNKS, CW = 100, 100
STAGE = 25
NBUF = 3
NP = 10240
RB = 1000
EP = NS * CHUNKS * CW


def _sc_mesh():
    return plsc.VectorSubcoreMesh(core_axis_name="c", subcore_axis_name="s")


def _degrees(edges_t, ones_np):
    @functools.partial(
        pl.kernel,
        out_type=jax.ShapeDtypeStruct((2 * NP,), jnp.float32),
        mesh=_sc_mesh(),
        scratch_types=[
            pltpu.VMEM((CHUNKS, CW), jnp.int32),
            pltpu.VMEM((CW,), jnp.float32),
            pltpu.SemaphoreType.DMA,
            pltpu.VMEM_SHARED((NP,), jnp.float32),
        ],
    )
    def k(edges_hbm, ones_hbm, deg_hbm, idx_v, ones_v, dsem, deg_sh):
        c = lax.axis_index("c")
        s = lax.axis_index("s")
        pltpu.sync_copy(edges_hbm.at[c * NS + s], idx_v)
        pltpu.sync_copy(ones_hbm.at[pl.ds(0, CW)], ones_v)

        @pl.when(s == 0)
        def _():
            pltpu.sync_copy(ones_hbm, deg_sh)

        plsc.subcore_barrier()

        @pl.loop(0, CHUNKS, step=10)
        def _(j0):
            for dj in range(10):
                pltpu.async_copy(ones_v, deg_sh.at[idx_v.at[j0 + dj]],
                                 dsem, add=True)
            for dj in range(10):
                pltpu.make_async_copy(ones_v, deg_sh.at[idx_v.at[0]],
                                      dsem).wait()

        plsc.subcore_barrier()

        @pl.when(s == 0)
        def _():
            pltpu.sync_copy(deg_sh, deg_hbm.at[pl.ds(c * NP, NP)])

    return k(edges_t, ones_np)


def _scatter_rows(vals_flat, src2_t, dst_t, d):
    rows_per = NP // NS

    @functools.partial(
        pl.kernel,
        out_type=jax.ShapeDtypeStruct((2 * NP, d), jnp.float32),
        mesh=_sc_mesh(),
        scratch_types=[
            pltpu.VMEM((STAGE, CW), jnp.int32),
            pltpu.VMEM((STAGE, CW), jnp.int32),
            pltpu.VMEM((NBUF, CW, d), jnp.float32),
            pltpu.SemaphoreType.DMA((NBUF,)),
            pltpu.SemaphoreType.DMA((NBUF,)),
            pltpu.VMEM_SHARED((NP, d), jnp.float32),
        ],
    )
    def k(vals_hbm, src_hbm, dst_hbm, out_hbm, src_v, dst_v, rows_v, gsem,
          ssem, acc_sh):
        c = lax.axis_index("c")
        s = lax.axis_index("s")
        base = s * rows_per
        pltpu.sync_copy(vals_hbm.at[pl.ds(c * NP + base, rows_per)],
                        acc_sh.at[pl.ds(base, rows_per)])
        plsc.subcore_barrier()

        def gather(j, b):
            pltpu.async_copy(vals_hbm.at[src_v.at[j]], rows_v.at[b],
                             gsem.at[b])

        def wait_gather(b):
            pltpu.make_async_copy(vals_hbm.at[src_v.at[0]], rows_v.at[b],
                                  gsem.at[b]).wait()

        def scatter(j, b):
            pltpu.async_copy(rows_v.at[b], acc_sh.at[dst_v.at[j]],
                             ssem.at[b], add=True)

        def wait_scatter(b):
            pltpu.make_async_copy(rows_v.at[b], acc_sh.at[dst_v.at[0]],
                                  ssem.at[b]).wait()

        nstage = CHUNKS // STAGE
        for st in range(nstage):
            pltpu.sync_copy(src_hbm.at[(c * NS + s) * nstage + st], src_v)
            pltpu.sync_copy(dst_hbm.at[s * nstage + st], dst_v)
            gather(0, 0)
            gather(1, 1)
            wait_gather(0)
            gather(2, 2)
            scatter(0, 0)

            @pl.loop(1, STAGE, step=NBUF)
            def _(j0):
                for dj in range(NBUF):
                    j = j0 + dj
                    b = (1 + dj) % NBUF
                    b2 = (b + 2) % NBUF
                    wait_gather(b)
                    wait_scatter(b2)

                    @pl.when(j + 2 < STAGE)
                    def _():
                        gather(j + 2, b2)

                    scatter(j, b)

            wait_scatter((STAGE - 1) % NBUF)

        plsc.subcore_barrier()
        pltpu.sync_copy(acc_sh.at[pl.ds(base, rows_per)],
                        out_hbm.at[pl.ds(c * NP + base, rows_per)])

    return k(vals_flat, src2_t, dst_t)


def _prep(x, deg_o):
    n, d = x.shape
    half = d // 2

    def body(x_ref, dg_ref, out_ref):
        do = lax.rsqrt(dg_ref[...])
        xs = x_ref[...] * do
        out_ref[0] = xs[:, :half]
        out_ref[1] = xs[:, half:]

    return pl.pallas_call(
        body,
        grid=(n // RB,),
        in_specs=[pl.BlockSpec((RB, d), lambda i: (i, 0)),
                  pl.BlockSpec((RB, 1), lambda i: (i, 0))],
        out_specs=pl.BlockSpec((2, RB, half), lambda i: (0, i, 0)),
        out_shape=jax.ShapeDtypeStruct((2, NP, half), jnp.float32),
    )(x, deg_o)


def _mm(agg2, deg_i, deg_o, W1, b1, W2):
    _, n, half = agg2.shape
    d_hid = W1.shape[1]

    def body(a_ref, di_ref, do_ref, w1_ref, b1_ref, w2_ref, out_ref):
        di = lax.rsqrt(di_ref[...])
        do = lax.rsqrt(do_ref[...])
        agg = jnp.concatenate([a_ref[0], a_ref[1]], axis=1)
        h1 = jnp.dot((agg * di).astype(jnp.bfloat16), w1_ref[...],
                     preferred_element_type=jnp.float32)
        h1 = jnp.maximum(h1 + b1_ref[...], 0.0)
        t = jnp.dot((h1 * do).astype(jnp.bfloat16), w2_ref[...],
                    preferred_element_type=jnp.float32)
        out_ref[0] = t[:, :half]
        out_ref[1] = t[:, half:]

    return pl.pallas_call(
        body,
        grid=(n // RB,),
        in_specs=[pl.BlockSpec((2, RB, half), lambda i: (0, i, 0)),
                  pl.BlockSpec((RB, 1), lambda i: (i, 0)),
                  pl.BlockSpec((RB, 1), lambda i: (i, 0)),
                  pl.BlockSpec(W1.shape, lambda i: (0, 0)),
                  pl.BlockSpec((1, d_hid), lambda i: (0, 0)),
                  pl.BlockSpec(W2.shape, lambda i: (0, 0))],
        out_specs=pl.BlockSpec((2, RB, half), lambda i: (0, i, 0)),
        out_shape=jax.ShapeDtypeStruct((2, n, half), jnp.float32),
    )(agg2, deg_i, deg_o, W1, b1, W2)


def _final(agg2, deg_i, b2, n_valid):
    d = 2 * agg2.shape[2]

    def body(a_ref, di_ref, b2_ref, out_ref):
        i = pl.program_id(0)
        di = lax.rsqrt(di_ref[...])
        agg = jnp.concatenate([a_ref[0], a_ref[1]], axis=1)
        h2 = jnp.maximum(agg * di + b2_ref[...], 0.0)
        part = jnp.sum(h2, axis=0, keepdims=True)

        @pl.when(i == 0)
        def _():
            out_ref[...] = jnp.zeros((1, d), jnp.float32)

        out_ref[...] += part

        @pl.when(i == pl.num_programs(0) - 1)
        def _():
            out_ref[...] = out_ref[...] * (1.0 / n_valid)

    return pl.pallas_call(
        body,
        grid=(n_valid // RB,),
        in_specs=[pl.BlockSpec((2, RB, d // 2), lambda i: (0, i, 0)),
                  pl.BlockSpec((RB, 1), lambda i: (i, 0)),
                  pl.BlockSpec((1, d), lambda i: (0, 0))],
        out_specs=pl.BlockSpec((1, d), lambda i: (0, 0)),
        out_shape=jax.ShapeDtypeStruct((1, d), jnp.float32),
    )(agg2, deg_i, b2)


def kernel(x, edge_index, W1, b1, W2, b2):
    n, d_in = x.shape
    half = d_in // 2

    src = edge_index[0]
    dst = edge_index[1]
    edges_t = edge_index.reshape(2 * NS, CHUNKS, CW)
    ones_np = jnp.ones((NP,), jnp.float32)

    deg = _degrees(edges_t, ones_np)
    deg_o = deg[:NP].reshape(NP, 1)
    deg_i = deg[NP:].reshape(NP, 1)

    nstage = CHUNKS // STAGE
    src2 = jnp.stack([src, src + NP]).reshape(2 * NS * nstage, STAGE, CW)
    dst_t = dst.reshape(NS * nstage, STAGE, CW)

    xs2 = _prep(x, deg_o)
    agg1 = _scatter_rows(xs2.reshape(2 * NP, half), src2, dst_t, half)
    t2 = _mm(agg1.reshape(2, NP, half), deg_i, deg_o,
             W1.astype(jnp.bfloat16), b1.reshape(1, -1),
             W2.astype(jnp.bfloat16))
    agg2 = _scatter_rows(t2.reshape(2 * NP, half), src2, dst_t, half)
    return _final(agg2.reshape(2, NP, half), deg_i, b2.reshape(1, -1), n)


# speedup vs baseline: 1.0410x; 1.0110x over previous
import functools

import jax
import jax.numpy as jnp
from jax import lax
from jax.experimental import pallas as pl
from jax.experimental.pallas import tpu as pltpu
from jax.experimental.pallas import tpu_sc as plsc

NC, NS = 2, 16
CHUNKS, CW = 100, 100
STAGE = 25
NBUF = 3
NP = 10240
RB = 1000
EP = NS * CHUNKS * CW


def _sc_mesh():
    return plsc.VectorSubcoreMesh(core_axis_name="c", subcore_axis_name="s")


def _degrees(edges_t, ones_np):
    @functools.partial(
        pl.kernel,
        out_type=jax.ShapeDtypeStruct((2 * NP,), jnp.float32),
        mesh=_sc_mesh(),
        scratch_types=[
            pltpu.VMEM((CHUNKS, CW), jnp.int32),
            pltpu.VMEM((CW,), jnp.float32),
            pltpu.SemaphoreType.DMA,
            pltpu.VMEM_SHARED((NP,), jnp.float32),
        ],
    )
    def k(edges_hbm, ones_hbm, deg_hbm, idx_v, ones_v, dsem, deg_sh):
        c = lax.axis_index("c")
        s = lax.axis_index("s")
        pltpu.sync_copy(edges_hbm.at[c * NS + s], idx_v)
        pltpu.sync_copy(ones_hbm.at[pl.ds(0, CW)], ones_v)

        @pl.when(s == 0)
        def _():
            pltpu.sync_copy(ones_hbm, deg_sh)

        plsc.subcore_barrier()

        @pl.loop(0, CHUNKS, step=10)
        def _(j0):
            for dj in range(10):
                pltpu.async_copy(ones_v, deg_sh.at[idx_v.at[j0 + dj]],
                                 dsem, add=True)
            for dj in range(10):
                pltpu.make_async_copy(ones_v, deg_sh.at[idx_v.at[0]],
                                      dsem).wait()

        plsc.subcore_barrier()

        @pl.when(s == 0)
        def _():
            pltpu.sync_copy(deg_sh, deg_hbm.at[pl.ds(c * NP, NP)])

    return k(edges_t, ones_np)


def _scatter_rows(vals_flat, src2_t, dst_t, d):
    rows_per = NP // NS

    @functools.partial(
        pl.kernel,
        out_type=jax.ShapeDtypeStruct((2 * NP, d), jnp.float32),
        mesh=_sc_mesh(),
        scratch_types=[
            pltpu.VMEM((STAGE, CW), jnp.int32),
            pltpu.VMEM((STAGE, CW), jnp.int32),
            pltpu.VMEM((NBUF, CW, d), jnp.float32),
            pltpu.SemaphoreType.DMA((NBUF,)),
            pltpu.SemaphoreType.DMA((NBUF,)),
            pltpu.VMEM_SHARED((NP, d), jnp.float32),
        ],
    )
    def k(vals_hbm, src_hbm, dst_hbm, out_hbm, src_v, dst_v, rows_v, gsem,
          ssem, acc_sh):
        c = lax.axis_index("c")
        s = lax.axis_index("s")
        base = s * rows_per
        pltpu.sync_copy(vals_hbm.at[pl.ds(c * NP + base, rows_per)],
                        acc_sh.at[pl.ds(base, rows_per)])

        def gather(j, b):
            pltpu.async_copy(vals_hbm.at[src_v.at[j]], rows_v.at[b],
                             gsem.at[b])

        def wait_gather(b):
            pltpu.make_async_copy(vals_hbm.at[src_v.at[0]], rows_v.at[b],
                                  gsem.at[b]).wait()

        def scatter(j, b):
            pltpu.async_copy(rows_v.at[b], acc_sh.at[dst_v.at[j]],
                             ssem.at[b], add=True)

        def wait_scatter(b):
            pltpu.make_async_copy(rows_v.at[b], acc_sh.at[dst_v.at[0]],
                                  ssem.at[b]).wait()

        nstage = CHUNKS // STAGE
        for st in range(nstage):
            if st == 0:
                pltpu.sync_copy(src_hbm.at[(c * NS + s) * nstage], src_v)
                pltpu.sync_copy(dst_hbm.at[s * nstage], dst_v)
                gather(0, 0)
                gather(1, 1)
                plsc.subcore_barrier()
            else:
                pltpu.sync_copy(src_hbm.at[(c * NS + s) * nstage + st],
                                src_v)
                pltpu.sync_copy(dst_hbm.at[s * nstage + st], dst_v)
                gather(0, 0)
                gather(1, 1)
            wait_gather(0)
            gather(2, 2)
            scatter(0, 0)

            @pl.loop(1, STAGE, step=NBUF)
            def _(j0):
                for dj in range(NBUF):
                    j = j0 + dj
                    b = (1 + dj) % NBUF
                    b2 = (b + 2) % NBUF
                    wait_gather(b)
                    wait_scatter(b2)

                    @pl.when(j + 2 < STAGE)
                    def _():
                        gather(j + 2, b2)

                    scatter(j, b)

            wait_scatter((STAGE - 1) % NBUF)

        plsc.subcore_barrier()
        pltpu.sync_copy(acc_sh.at[pl.ds(base, rows_per)],
                        out_hbm.at[pl.ds(c * NP + base, rows_per)])

    return k(vals_flat, src2_t, dst_t)


def _prep(x, deg_o):
    n, d = x.shape
    half = d // 2

    def body(x_ref, dg_ref, out_ref):
        do = lax.rsqrt(dg_ref[...])
        xs = x_ref[...] * do
        out_ref[0] = xs[:, :half]
        out_ref[1] = xs[:, half:]

    return pl.pallas_call(
        body,
        grid=(n // RB,),
        in_specs=[pl.BlockSpec((RB, d), lambda i: (i, 0)),
                  pl.BlockSpec((RB, 1), lambda i: (i, 0))],
        out_specs=pl.BlockSpec((2, RB, half), lambda i: (0, i, 0)),
        out_shape=jax.ShapeDtypeStruct((2, NP, half), jnp.float32),
    )(x, deg_o)


def _mm(agg2, deg_i, deg_o, W1, b1, W2):
    _, n, half = agg2.shape
    d_hid = W1.shape[1]

    def body(a_ref, di_ref, do_ref, w1_ref, b1_ref, w2_ref, out_ref):
        di = lax.rsqrt(di_ref[...])
        do = lax.rsqrt(do_ref[...])
        agg = jnp.concatenate([a_ref[0], a_ref[1]], axis=1)
        h1 = jnp.dot((agg * di).astype(jnp.bfloat16), w1_ref[...],
                     preferred_element_type=jnp.float32)
        h1 = jnp.maximum(h1 + b1_ref[...], 0.0)
        t = jnp.dot((h1 * do).astype(jnp.bfloat16), w2_ref[...],
                    preferred_element_type=jnp.float32)
        out_ref[0] = t[:, :half]
        out_ref[1] = t[:, half:]

    return pl.pallas_call(
        body,
        grid=(n // RB,),
        in_specs=[pl.BlockSpec((2, RB, half), lambda i: (0, i, 0)),
                  pl.BlockSpec((RB, 1), lambda i: (i, 0)),
                  pl.BlockSpec((RB, 1), lambda i: (i, 0)),
                  pl.BlockSpec(W1.shape, lambda i: (0, 0)),
                  pl.BlockSpec((1, d_hid), lambda i: (0, 0)),
                  pl.BlockSpec(W2.shape, lambda i: (0, 0))],
        out_specs=pl.BlockSpec((2, RB, half), lambda i: (0, i, 0)),
        out_shape=jax.ShapeDtypeStruct((2, n, half), jnp.float32),
    )(agg2, deg_i, deg_o, W1, b1, W2)


def _final(agg2, deg_i, b2, n_valid):
    d = 2 * agg2.shape[2]

    def body(a_ref, di_ref, b2_ref, out_ref):
        i = pl.program_id(0)
        di = lax.rsqrt(di_ref[...])
        agg = jnp.concatenate([a_ref[0], a_ref[1]], axis=1)
        h2 = jnp.maximum(agg * di + b2_ref[...], 0.0)
        part = jnp.sum(h2, axis=0, keepdims=True)

        @pl.when(i == 0)
        def _():
            out_ref[...] = jnp.zeros((1, d), jnp.float32)

        out_ref[...] += part

        @pl.when(i == pl.num_programs(0) - 1)
        def _():
            out_ref[...] = out_ref[...] * (1.0 / n_valid)

    return pl.pallas_call(
        body,
        grid=(n_valid // RB,),
        in_specs=[pl.BlockSpec((2, RB, d // 2), lambda i: (0, i, 0)),
                  pl.BlockSpec((RB, 1), lambda i: (i, 0)),
                  pl.BlockSpec((1, d), lambda i: (0, 0))],
        out_specs=pl.BlockSpec((1, d), lambda i: (0, 0)),
        out_shape=jax.ShapeDtypeStruct((1, d), jnp.float32),
    )(agg2, deg_i, b2)


def kernel(x, edge_index, W1, b1, W2, b2):
    n, d_in = x.shape
    half = d_in // 2

    src = edge_index[0]
    dst = edge_index[1]
    edges_t = edge_index.reshape(2 * NS, CHUNKS, CW)
    ones_np = jnp.ones((NP,), jnp.float32)

    deg = _degrees(edges_t, ones_np)
    deg_o = deg[:NP].reshape(NP, 1)
    deg_i = deg[NP:].reshape(NP, 1)

    nstage = CHUNKS // STAGE
    src2 = jnp.stack([src, src + NP]).reshape(2 * NS * nstage, STAGE, CW)
    dst_t = dst.reshape(NS * nstage, STAGE, CW)

    xs2 = _prep(x, deg_o)
    agg1 = _scatter_rows(xs2.reshape(2 * NP, half), src2, dst_t, half)
    t2 = _mm(agg1.reshape(2, NP, half), deg_i, deg_o,
             W1.astype(jnp.bfloat16), b1.reshape(1, -1),
             W2.astype(jnp.bfloat16))
    agg2 = _scatter_rows(t2.reshape(2 * NP, half), src2, dst_t, half)
    return _final(agg2.reshape(2, NP, half), deg_i, b2.reshape(1, -1), n)
